# baseline (speedup 1.0000x reference)
import functools

import jax
import jax.numpy as jnp
from jax import lax
from jax.experimental import pallas as pl
from jax.experimental.pallas import tpu as pltpu
from jax.experimental.pallas import tpu_sc as plsc

BATCH = 16384
EMB = 32
NENT = 1000000
NREL = 1000
NC = 2
NS = 16
NW = NC * NS
BPW = BATCH // NW
LANES = 16
GROUPS = BPW // LANES
DCH = 4
NROUND = EMB // DCH
GR = DCH * BPW

_mesh = plsc.VectorSubcoreMesh(core_axis_name="c", subcore_axis_name="s")


@functools.partial(
    pl.kernel,
    mesh=_mesh,
    out_type=jax.ShapeDtypeStruct((BATCH,), jnp.float32),
    scratch_types=[
        pltpu.VMEM((BPW,), jnp.int32),
        pltpu.VMEM((BPW,), jnp.int32),
        pltpu.VMEM((BPW,), jnp.int32),
        pltpu.VMEM((GR,), jnp.int32),
        pltpu.VMEM((GR,), jnp.int32),
        pltpu.VMEM((GR,), jnp.int32),
        pltpu.VMEM((GR, 16), jnp.float32),
        pltpu.VMEM((GR, 16), jnp.float32),
        pltpu.VMEM((GR, 16), jnp.float32),
        pltpu.VMEM((BPW,), jnp.float32),
        pltpu.SemaphoreType.DMA,
    ],
    compiler_params=pltpu.CompilerParams(
        needs_layout_passes=False, use_tc_tiling_on_sc=False
    ),
)
def _transe_sc(h_hbm, r_hbm, t_hbm, ent_hbm, rel_hbm, out_hbm,
               hi, ri, ti, hx, rx, tx, hg, rg, tg, ov, sem):
    wid = lax.axis_index("s") * NC + lax.axis_index("c")
    base = wid * BPW

    pltpu.sync_copy(h_hbm.at[pl.ds(base, BPW)], hi)
    pltpu.sync_copy(r_hbm.at[pl.ds(base, BPW)], ri)
    pltpu.sync_copy(t_hbm.at[pl.ds(base, BPW)], ti)

    lanes = lax.iota(jnp.int32, LANES)

    for c in range(NROUND):
        def idx_body(g, carry, c=c):
            i0 = g * LANES
            he = hi[pl.ds(i0, LANES)]
            re = ri[pl.ds(i0, LANES)]
            te = ti[pl.ds(i0, LANES)]
            for dl in range(DCH):
                d = c * DCH + dl
                dst = pl.ds(dl * BPW + i0, LANES)
                hx[dst] = d * (NENT // 16) + (he >> 4)
                rx[dst] = (d * NREL + re) >> 4
                tx[dst] = d * (NENT // 16) + (te >> 4)
            return carry

        lax.fori_loop(0, GROUPS, idx_body, 0)

        cp_h = pltpu.async_copy(ent_hbm.at[hx], hg, sem)
        cp_r = pltpu.async_copy(rel_hbm.at[rx], rg, sem)
        cp_t = pltpu.async_copy(ent_hbm.at[tx], tg, sem)
        cp_h.wait()
        cp_r.wait()
        cp_t.wait()

        def acc_body(g, carry, c=c):
            i0 = g * LANES
            he = hi[pl.ds(i0, LANES)]
            re = ri[pl.ds(i0, LANES)]
            te = ti[pl.ds(i0, LANES)]
            ch = he & 15
            ct = te & 15
            acc = jnp.zeros((LANES,), jnp.float32) if c == 0 else ov[pl.ds(i0, LANES)]
            for dl in range(DCH):
                d = c * DCH + dl
                cr = (d * NREL + re) & 15
                rows = dl * BPW + i0 + lanes
                hh = plsc.load_gather(hg, [rows, ch])
                rr = plsc.load_gather(rg, [rows, cr])
                tt = plsc.load_gather(tg, [rows, ct])
                acc = acc + jnp.abs(hh + rr - tt)
            ov[pl.ds(i0, LANES)] = acc
            return carry

        lax.fori_loop(0, GROUPS, acc_body, 0)

    pltpu.sync_copy(ov, out_hbm.at[pl.ds(base, BPW)])


def kernel(batch_h, batch_r, batch_t, entity_embds, rel_embds):
    ent_g = jnp.reshape(entity_embds.T, (EMB * NENT // 16, 16))
    rel_g = jnp.reshape(rel_embds.T, (EMB * NREL // 16, 16))
    return _transe_sc(batch_h, batch_r, batch_t, ent_g, rel_g)


# speedup vs baseline: 5.1953x; 5.1953x over previous
import functools

import jax
import jax.numpy as jnp
from jax import lax
from jax.experimental import pallas as pl
from jax.experimental.pallas import tpu as pltpu
from jax.experimental.pallas import tpu_sc as plsc

BATCH = 16384
EMB = 32
NC = 2
NS = 16
NW = NC * NS
BPW = BATCH // NW
LANES = 16
GROUPS = BPW // LANES

_mesh = plsc.VectorSubcoreMesh(core_axis_name="c", subcore_axis_name="s")


@functools.partial(
    pl.kernel,
    mesh=_mesh,
    out_type=jax.ShapeDtypeStruct((BATCH,), jnp.float32),
    scratch_types=[
        pltpu.VMEM((2 * BPW,), jnp.int32),
        pltpu.VMEM((BPW,), jnp.int32),
        pltpu.VMEM((2 * BPW, EMB), jnp.float32),
        pltpu.VMEM((BPW, EMB), jnp.float32),
        pltpu.VMEM((BPW,), jnp.float32),
        pltpu.SemaphoreType.DMA,
    ],
    compiler_params=pltpu.CompilerParams(
        needs_layout_passes=False, use_tc_tiling_on_sc=False
    ),
)
def _transe_sc(h_hbm, r_hbm, t_hbm, ent_hbm, rel_hbm, out_hbm,
               hti, ri, htv, rv, ov, sem):
    wid = lax.axis_index("s") * NC + lax.axis_index("c")
    base = wid * BPW

    pltpu.sync_copy(h_hbm.at[pl.ds(base, BPW)], hti.at[pl.ds(0, BPW)])
    pltpu.sync_copy(t_hbm.at[pl.ds(base, BPW)], hti.at[pl.ds(BPW, BPW)])
    pltpu.sync_copy(r_hbm.at[pl.ds(base, BPW)], ri)

    cp_ht = pltpu.async_copy(ent_hbm.at[hti], htv, sem)
    cp_r = pltpu.async_copy(rel_hbm.at[ri], rv, sem)
    cp_ht.wait()
    cp_r.wait()

    lanes = lax.iota(jnp.int32, LANES)

    def group_body(g, carry):
        acc = jnp.zeros((LANES,), jnp.float32)
        for i in range(LANES):
            row = g * LANES + i
            h0 = htv[row, pl.ds(0, LANES)]
            h1 = htv[row, pl.ds(LANES, LANES)]
            t0 = htv[BPW + row, pl.ds(0, LANES)]
            t1 = htv[BPW + row, pl.ds(LANES, LANES)]
            r0 = rv[row, pl.ds(0, LANES)]
            r1 = rv[row, pl.ds(LANES, LANES)]
            e = jnp.abs(h0 + r0 - t0) + jnp.abs(h1 + r1 - t1)
            s = jnp.sum(e)
            acc = jnp.where(lanes == i, s, acc)
        ov[pl.ds(pl.multiple_of(g * LANES, LANES), LANES)] = acc
        return carry

    lax.fori_loop(0, GROUPS, group_body, 0)

    pltpu.sync_copy(ov, out_hbm.at[pl.ds(base, BPW)])


def kernel(batch_h, batch_r, batch_t, entity_embds, rel_embds):
    return _transe_sc(batch_h, batch_r, batch_t, entity_embds, rel_embds)
